# baseline (speedup 1.0000x reference)
import jax
import jax.numpy as jnp
from jax import lax
from jax.experimental import pallas as pl
from jax.experimental.pallas import tpu as pltpu
from jax.experimental.pallas import tpu_sc as plsc

NUM_NODES = 100000
HIDDEN_DIM = 128
NUM_GRAPHS = 512

NC = 2
NS = 16
NW = NC * NS
GPW = NUM_GRAPHS // NW
LANES = 16
FPR = HIDDEN_DIM // LANES

CH = 240
EFF = CH - 8
NCAP = NUM_NODES - CH

NGROUPS = NUM_NODES // LANES
GQ, GR = divmod(NGROUPS, NW)
SCG = 8
SCROWS = SCG * LANES

NEG_INF = float("-inf")
BIGF = 3.402823466e38


def _bcast(s):
  return lax.broadcast_in_dim(s, (LANES,), ())


def _seg_kernel(x_hbm, batch_hbm, starts_hbm, emb_out, cnt_out,
                xbuf, gmaxbuf, cntbuf, startsbuf):
  del batch_hbm
  wid = lax.axis_index("c") * NS + lax.axis_index("s")
  g_base = wid * GPW

  pltpu.sync_copy(starts_hbm, startsbuf)

  iota = lax.iota(jnp.int32, LANES)
  perms = [jnp.bitwise_xor(iota, 1 << k) for k in range(4)]

  iota_f = jnp.asarray(iota, jnp.float32)

  def per_graph(i, cnt_vec):
    g = g_base + i
    sl = startsbuf[pl.ds(g, LANES)]
    s = sl[0]
    e = sl[1]
    nk = lax.div(e - s + (EFF - 1), EFF)

    def p1_chunk(k, acc):
      pos = s + k * EFF
      base = pl.multiple_of(
          jnp.bitwise_and(jnp.minimum(pos, NCAP), -8), 8)
      pltpu.sync_copy(x_hbm.at[pl.ds(base, CH)], xbuf)
      lo = pos - base
      hi = jnp.minimum(pos + EFF, e) - base

      def p1_row(j, acc):
        out = []
        for c in range(FPR):
          xv = xbuf[j, pl.ds(c * LANES, LANES)]
          out.append(jnp.maximum(acc[c], xv))
        return tuple(out)

      return lax.fori_loop(lo, hi, p1_row, acc)

    acc0 = tuple(jnp.full((LANES,), NEG_INF, jnp.float32) for _ in range(FPR))
    acc = lax.fori_loop(0, nk, p1_chunk, acc0)
    for c in range(FPR):
      gmaxbuf[i, pl.ds(c * LANES, LANES)] = acc[c]

    def p2_chunk(k, cnt):
      pos = s + k * EFF
      base = pl.multiple_of(
          jnp.bitwise_and(jnp.minimum(pos, NCAP), -8), 8)
      @pl.when(jnp.logical_or(k > 0, nk > 1))
      def _():
        pltpu.sync_copy(x_hbm.at[pl.ds(base, CH)], xbuf)
      lo = pos - base
      hi = jnp.minimum(pos + EFF, e) - base

      def p2_row(j, cnt):
        d = jnp.full((LANES,), -BIGF, jnp.float32)
        for c in range(FPR):
          xv = xbuf[j, pl.ds(c * LANES, LANES)]
          d = jnp.maximum(d, xv - acc[c])
        for p in perms:
          d = jnp.maximum(d, d[p])
        return cnt + (1.0 - jnp.minimum(-d * BIGF * BIGF, 1.0))

      return lax.fori_loop(lo, hi, p2_row, cnt)

    cnt = lax.fori_loop(0, nk, p2_chunk, jnp.zeros((LANES,), jnp.float32))
    cnt = jnp.maximum(cnt, 1.0)
    onehot = jnp.maximum(1.0 - jnp.abs(iota_f - _bcast(i.astype(jnp.float32))),
                         0.0)
    return cnt_vec + onehot * cnt

  cnt_vec = lax.fori_loop(0, GPW, per_graph, jnp.zeros((LANES,), jnp.float32))
  cntbuf[...] = cnt_vec
  gb = pl.multiple_of(g_base, 8)
  pltpu.sync_copy(gmaxbuf, emb_out.at[pl.ds(gb, GPW)])
  pltpu.sync_copy(cntbuf, cnt_out.at[pl.ds(gb, GPW)])


def _score_kernel(x_hbm, batch_hbm, emb_hbm, cnt_hbm, scores_out,
                  xbuf, bbuf, embx, cntbuf, scorebuf, sem):
  wid = lax.axis_index("c") * NS + lax.axis_index("s")
  g0 = wid * GQ + jnp.minimum(wid, GR)
  ng = GQ + jnp.where(wid < GR, 1, 0)
  nsc = lax.div(ng + (SCG - 1), SCG)

  pltpu.sync_copy(cnt_hbm, cntbuf)

  iota = lax.iota(jnp.int32, LANES)
  iota_f = jnp.asarray(iota, jnp.float32)
  onehots = [
      jnp.maximum(1.0 - jnp.abs(iota_f - float(r)), 0.0) for r in range(LANES)]

  def per_chunk(k, carry):
    rbase = pl.multiple_of(
        jnp.minimum(g0 * LANES + k * SCROWS, NUM_NODES - SCROWS), 16)
    pltpu.sync_copy(x_hbm.at[pl.ds(rbase, SCROWS)], xbuf)
    pltpu.sync_copy(batch_hbm.at[pl.ds(rbase, SCROWS)], bbuf)
    pltpu.async_copy(emb_hbm.at[bbuf], embx, sem).wait()

    def per_group(j, carry):
      row0 = j * LANES
      bvec = bbuf[pl.ds(row0, LANES)]
      cv = plsc.load_gather(cntbuf, [bvec])
      svec = jnp.zeros((LANES,), jnp.float32)
      for r in range(LANES):
        d = jnp.full((LANES,), -BIGF, jnp.float32)
        for c in range(FPR):
          xv = xbuf[row0 + r, pl.ds(c * LANES, LANES)]
          ev = embx[row0 + r, pl.ds(c * LANES, LANES)]
          d = jnp.maximum(d, xv - ev)
        for p in perms:
          d = jnp.maximum(d, d[p])
        match = 1.0 - jnp.minimum(-d * BIGF * BIGF, 1.0)
        svec = svec + onehots[r] * match
      scorebuf[pl.ds(row0, LANES)] = svec / cv
      return carry

    lax.fori_loop(0, SCG, per_group, carry)
    pltpu.sync_copy(scorebuf, scores_out.at[pl.ds(rbase, SCROWS)])
    return carry

  perms = [jnp.bitwise_xor(iota, 1 << k) for k in range(4)]
  lax.fori_loop(0, nsc, per_chunk, jnp.int32(0))


def _mesh():
  return plsc.VectorSubcoreMesh(
      core_axis_name="c", subcore_axis_name="s", num_cores=NC, num_subcores=NS)


@jax.jit
def kernel(x, batch):
  batch = batch.astype(jnp.int32)
  starts = jnp.searchsorted(batch, jnp.arange(NUM_GRAPHS + 1, dtype=jnp.int32)
                            ).astype(jnp.int32)
  starts = jnp.concatenate(
      [starts, jnp.full((15,), NUM_NODES, jnp.int32)])

  seg = pl.kernel(
      _seg_kernel,
      out_type=(
          jax.ShapeDtypeStruct((NUM_GRAPHS, HIDDEN_DIM), jnp.float32),
          jax.ShapeDtypeStruct((NUM_GRAPHS,), jnp.float32),
      ),
      mesh=_mesh(),
      compiler_params=pltpu.CompilerParams(needs_layout_passes=False),
      scratch_types=[
          pltpu.VMEM((CH, HIDDEN_DIM), jnp.float32),
          pltpu.VMEM((GPW, HIDDEN_DIM), jnp.float32),
          pltpu.VMEM((GPW,), jnp.float32),
          pltpu.VMEM((NUM_GRAPHS + 16,), jnp.int32),
      ],
  )
  emb, cnt = seg(x, batch, starts)

  score = pl.kernel(
      _score_kernel,
      out_type=jax.ShapeDtypeStruct((NUM_NODES,), jnp.float32),
      mesh=_mesh(),
      compiler_params=pltpu.CompilerParams(needs_layout_passes=False),
      scratch_types=[
          pltpu.VMEM((SCROWS, HIDDEN_DIM), jnp.float32),
          pltpu.VMEM((SCROWS,), jnp.int32),
          pltpu.VMEM((SCROWS, HIDDEN_DIM), jnp.float32),
          pltpu.VMEM((NUM_GRAPHS,), jnp.float32),
          pltpu.VMEM((SCROWS,), jnp.float32),
          pltpu.SemaphoreType.DMA,
      ],
  )
  scores = score(x, batch, emb, cnt)
  return (emb, scores)


# speedup vs baseline: 1.7783x; 1.7783x over previous
import jax
import jax.numpy as jnp
from jax import lax
from jax.experimental import pallas as pl
from jax.experimental.pallas import tpu as pltpu
from jax.experimental.pallas import tpu_sc as plsc

NUM_NODES = 100000
HIDDEN_DIM = 128
NUM_GRAPHS = 512

NC = 2
NS = 16
NW = NC * NS
GPW = NUM_GRAPHS // NW
LANES = 16
FPR = HIDDEN_DIM // LANES

CH = 240
EFF = CH - 8
NCAP = NUM_NODES - CH

NGROUPS = NUM_NODES // LANES
GQ, GR = divmod(NGROUPS, NW)
SCG = 8
SCROWS = SCG * LANES

NEG_INF = float("-inf")
BIGF = 3.402823466e38


def _bcast(s):
  return lax.broadcast_in_dim(s, (LANES,), ())


def _seg_kernel(x_hbm, batch_hbm, starts_hbm, emb_out, cnt_out,
                xbuf, gmaxbuf, cntbuf, startsbuf):
  del batch_hbm
  wid = lax.axis_index("c") * NS + lax.axis_index("s")
  g_base = wid * GPW

  pltpu.sync_copy(starts_hbm, startsbuf)

  iota = lax.iota(jnp.int32, LANES)
  perms = [jnp.bitwise_xor(iota, 1 << k) for k in range(4)]

  iota_f = jnp.asarray(iota, jnp.float32)

  def per_graph(i, cnt_vec):
    g = g_base + i
    sl = startsbuf[pl.ds(g, LANES)]
    s = sl[0]
    e = sl[1]
    nk = lax.div(e - s + (EFF - 1), EFF)

    def p1_chunk(k, acc):
      pos = s + k * EFF
      base = pl.multiple_of(
          jnp.bitwise_and(jnp.minimum(pos, NCAP), -8), 8)
      pltpu.sync_copy(x_hbm.at[pl.ds(base, CH)], xbuf)
      lo = pos - base
      hi = jnp.minimum(pos + EFF, e) - base

      def p1_row(j, acc):
        out = []
        for c in range(FPR):
          xv = xbuf[j, pl.ds(c * LANES, LANES)]
          out.append(jnp.maximum(acc[c], xv))
        return tuple(out)

      return lax.fori_loop(lo, hi, p1_row, acc)

    acc0 = tuple(jnp.full((LANES,), NEG_INF, jnp.float32) for _ in range(FPR))
    acc = lax.fori_loop(0, nk, p1_chunk, acc0)
    for c in range(FPR):
      gmaxbuf[i, pl.ds(c * LANES, LANES)] = acc[c]

    def p2_chunk(k, cnt):
      pos = s + k * EFF
      base = pl.multiple_of(
          jnp.bitwise_and(jnp.minimum(pos, NCAP), -8), 8)
      @pl.when(jnp.logical_or(k > 0, nk > 1))
      def _():
        pltpu.sync_copy(x_hbm.at[pl.ds(base, CH)], xbuf)
      lo = pos - base
      hi = jnp.minimum(pos + EFF, e) - base

      def p2_row(j, cnt):
        d = jnp.full((LANES,), -BIGF, jnp.float32)
        for c in range(FPR):
          xv = xbuf[j, pl.ds(c * LANES, LANES)]
          d = jnp.maximum(d, xv - acc[c])
        for p in perms:
          d = jnp.maximum(d, d[p])
        return cnt + (1.0 - jnp.minimum(-d * BIGF * BIGF, 1.0))

      return lax.fori_loop(lo, hi, p2_row, cnt)

    cnt = lax.fori_loop(0, nk, p2_chunk, jnp.zeros((LANES,), jnp.float32))
    cnt = jnp.maximum(cnt, 1.0)
    onehot = jnp.maximum(1.0 - jnp.abs(iota_f - _bcast(i.astype(jnp.float32))),
                         0.0)
    return cnt_vec + onehot * cnt

  cnt_vec = lax.fori_loop(0, GPW, per_graph, jnp.zeros((LANES,), jnp.float32))
  cntbuf[...] = cnt_vec
  gb = pl.multiple_of(g_base, 8)
  pltpu.sync_copy(gmaxbuf, emb_out.at[pl.ds(gb, GPW)])
  pltpu.sync_copy(cntbuf, cnt_out.at[pl.ds(gb, GPW)])


def _score_kernel(x_hbm, batch_hbm, emb_hbm, cnt_hbm, scores_out,
                  xbuf, bbuf, embbuf, cntbuf, scorebuf):
  wid = lax.axis_index("c") * NS + lax.axis_index("s")
  g0 = wid * GQ + jnp.minimum(wid, GR)
  ng = GQ + jnp.where(wid < GR, 1, 0)
  nsc = lax.div(ng + (SCG - 1), SCG)

  pltpu.sync_copy(emb_hbm, embbuf)
  pltpu.sync_copy(cnt_hbm, cntbuf)

  iota = lax.iota(jnp.int32, LANES)
  iota_f = jnp.asarray(iota, jnp.float32)
  onehots = [
      jnp.maximum(1.0 - jnp.abs(iota_f - float(r)), 0.0) for r in range(LANES)]
  perms = [jnp.bitwise_xor(iota, 1 << k) for k in range(4)]

  def per_chunk(k, carry):
    rbase = pl.multiple_of(
        jnp.minimum(g0 * LANES + k * SCROWS, NUM_NODES - SCROWS), 16)
    pltpu.sync_copy(x_hbm.at[pl.ds(rbase, SCROWS)], xbuf)
    pltpu.sync_copy(batch_hbm.at[pl.ds(rbase, SCROWS)], bbuf)

    def per_group(j, carry):
      row0 = j * LANES
      bvec = bbuf[pl.ds(row0, LANES)]
      cv = plsc.load_gather(cntbuf, [bvec])
      svec = jnp.zeros((LANES,), jnp.float32)
      for r in range(LANES):
        gid = bvec[r]
        d = jnp.full((LANES,), -BIGF, jnp.float32)
        for c in range(FPR):
          xv = xbuf[row0 + r, pl.ds(c * LANES, LANES)]
          ev = embbuf[gid, pl.ds(c * LANES, LANES)]
          d = jnp.maximum(d, xv - ev)
        for p in perms:
          d = jnp.maximum(d, d[p])
        match = 1.0 - jnp.minimum(-d * BIGF * BIGF, 1.0)
        svec = svec + onehots[r] * match
      scorebuf[pl.ds(row0, LANES)] = svec / cv
      return carry

    lax.fori_loop(0, SCG, per_group, carry)
    pltpu.sync_copy(scorebuf, scores_out.at[pl.ds(rbase, SCROWS)])
    return carry

  lax.fori_loop(0, nsc, per_chunk, jnp.int32(0))


def _mesh():
  return plsc.VectorSubcoreMesh(
      core_axis_name="c", subcore_axis_name="s", num_cores=NC, num_subcores=NS)


@jax.jit
def kernel(x, batch):
  batch = batch.astype(jnp.int32)
  starts = jnp.searchsorted(batch, jnp.arange(NUM_GRAPHS + 1, dtype=jnp.int32)
                            ).astype(jnp.int32)
  starts = jnp.concatenate(
      [starts, jnp.full((15,), NUM_NODES, jnp.int32)])

  seg = pl.kernel(
      _seg_kernel,
      out_type=(
          jax.ShapeDtypeStruct((NUM_GRAPHS, HIDDEN_DIM), jnp.float32),
          jax.ShapeDtypeStruct((NUM_GRAPHS,), jnp.float32),
      ),
      mesh=_mesh(),
      compiler_params=pltpu.CompilerParams(needs_layout_passes=False),
      scratch_types=[
          pltpu.VMEM((CH, HIDDEN_DIM), jnp.float32),
          pltpu.VMEM((GPW, HIDDEN_DIM), jnp.float32),
          pltpu.VMEM((GPW,), jnp.float32),
          pltpu.VMEM((NUM_GRAPHS + 16,), jnp.int32),
      ],
  )
  emb, cnt = seg(x, batch, starts)

  score = pl.kernel(
      _score_kernel,
      out_type=jax.ShapeDtypeStruct((NUM_NODES,), jnp.float32),
      mesh=_mesh(),
      compiler_params=pltpu.CompilerParams(needs_layout_passes=False),
      scratch_types=[
          pltpu.VMEM((SCROWS, HIDDEN_DIM), jnp.float32),
          pltpu.VMEM((SCROWS,), jnp.int32),
          pltpu.VMEM((NUM_GRAPHS, HIDDEN_DIM), jnp.float32),
          pltpu.VMEM((NUM_GRAPHS,), jnp.float32),
          pltpu.VMEM((SCROWS,), jnp.float32),
      ],
  )
  scores = score(x, batch, emb, cnt)
  return (emb, scores)


# speedup vs baseline: 1.7861x; 1.0043x over previous
import jax
import jax.numpy as jnp
from jax import lax
from jax.experimental import pallas as pl
from jax.experimental.pallas import tpu as pltpu
from jax.experimental.pallas import tpu_sc as plsc

NUM_NODES = 100000
HIDDEN_DIM = 128
NUM_GRAPHS = 512

NC = 2
NS = 16
NW = NC * NS
GPW = NUM_GRAPHS // NW
LANES = 16
FPR = HIDDEN_DIM // LANES

CH = 240
EFF = CH - 8
NCAP = NUM_NODES - CH

NGROUPS = NUM_NODES // LANES
GQ, GR = divmod(NGROUPS, NW)
SCG = 8
SCROWS = SCG * LANES

NEG_INF = float("-inf")
BIGF = 3.402823466e38


def _bcast(s):
  return lax.broadcast_in_dim(s, (LANES,), ())


def _seg_kernel(x_hbm, batch2d_hbm, emb_out, cnt_out,
                xbuf, gmaxbuf, cntbuf, d1buf, d2buf, sem1, sem2):
  wid = lax.axis_index("c") * NS + lax.axis_index("s")
  g_base = wid * GPW

  iota = lax.iota(jnp.int32, LANES)
  perms = [jnp.bitwise_xor(iota, 1 << k) for k in range(4)]

  iota_f = jnp.asarray(iota, jnp.float32)

  tgt1 = g_base + iota
  tgt2 = tgt1 + 1

  def bs_step(t, carry):
    lo1, hi1, lo2, hi2 = carry
    mid1 = lax.shift_right_arithmetic(lo1 + hi1, 1)
    mid2 = lax.shift_right_arithmetic(lo2 + hi2, 1)
    m1 = jnp.minimum(mid1, NUM_NODES - 1)
    m2 = jnp.minimum(mid2, NUM_NODES - 1)
    cp1 = pltpu.async_copy(
        batch2d_hbm.at[lax.shift_right_arithmetic(m1, 7)], d1buf, sem1)
    cp2 = pltpu.async_copy(
        batch2d_hbm.at[lax.shift_right_arithmetic(m2, 7)], d2buf, sem2)
    cp1.wait()
    cp2.wait()
    b1 = plsc.load_gather(d1buf, [iota, jnp.bitwise_and(m1, 127)])
    b2 = plsc.load_gather(d2buf, [iota, jnp.bitwise_and(m2, 127)])
    lo1 = jnp.where(b1 < tgt1, mid1 + 1, lo1)
    hi1 = jnp.where(b1 < tgt1, hi1, mid1)
    lo2 = jnp.where(b2 < tgt2, mid2 + 1, lo2)
    hi2 = jnp.where(b2 < tgt2, hi2, mid2)
    return (lo1, hi1, lo2, hi2)

  z = jnp.zeros((LANES,), jnp.int32)
  nfull = jnp.full((LANES,), NUM_NODES, jnp.int32)
  lo1, _, lo2, _ = lax.fori_loop(0, 17, bs_step, (z, nfull, z, nfull))
  starts_vec = jnp.minimum(lo1, NUM_NODES)
  ends_vec = jnp.minimum(lo2, NUM_NODES)

  def per_graph(i, cnt_vec):
    bi = _bcast(i)
    s = starts_vec[bi][0]
    e = ends_vec[bi][0]
    nk = lax.div(e - s + (EFF - 1), EFF)

    def p1_chunk(k, acc):
      pos = s + k * EFF
      base = pl.multiple_of(
          jnp.bitwise_and(jnp.minimum(pos, NCAP), -8), 8)
      pltpu.sync_copy(x_hbm.at[pl.ds(base, CH)], xbuf)
      lo = pos - base
      hi = jnp.minimum(pos + EFF, e) - base

      def p1_row(j, acc):
        out = []
        for c in range(FPR):
          xv = xbuf[j, pl.ds(c * LANES, LANES)]
          out.append(jnp.maximum(acc[c], xv))
        return tuple(out)

      return lax.fori_loop(lo, hi, p1_row, acc)

    acc0 = tuple(jnp.full((LANES,), NEG_INF, jnp.float32) for _ in range(FPR))
    acc = lax.fori_loop(0, nk, p1_chunk, acc0)
    for c in range(FPR):
      gmaxbuf[i, pl.ds(c * LANES, LANES)] = acc[c]

    def p2_chunk(k, cnt):
      pos = s + k * EFF
      base = pl.multiple_of(
          jnp.bitwise_and(jnp.minimum(pos, NCAP), -8), 8)
      @pl.when(jnp.logical_or(k > 0, nk > 1))
      def _():
        pltpu.sync_copy(x_hbm.at[pl.ds(base, CH)], xbuf)
      lo = pos - base
      hi = jnp.minimum(pos + EFF, e) - base

      def p2_row(j, cnt):
        d = jnp.full((LANES,), -BIGF, jnp.float32)
        for c in range(FPR):
          xv = xbuf[j, pl.ds(c * LANES, LANES)]
          d = jnp.maximum(d, xv - acc[c])
        for p in perms:
          d = jnp.maximum(d, d[p])
        return cnt + (1.0 - jnp.minimum(-d * BIGF * BIGF, 1.0))

      return lax.fori_loop(lo, hi, p2_row, cnt)

    cnt = lax.fori_loop(0, nk, p2_chunk, jnp.zeros((LANES,), jnp.float32))
    cnt = jnp.maximum(cnt, 1.0)
    onehot = jnp.maximum(1.0 - jnp.abs(iota_f - _bcast(i.astype(jnp.float32))),
                         0.0)
    return cnt_vec + onehot * cnt

  cnt_vec = lax.fori_loop(0, GPW, per_graph, jnp.zeros((LANES,), jnp.float32))
  cntbuf[...] = cnt_vec
  gb = pl.multiple_of(g_base, 8)
  pltpu.sync_copy(gmaxbuf, emb_out.at[pl.ds(gb, GPW)])
  pltpu.sync_copy(cntbuf, cnt_out.at[pl.ds(gb, GPW)])


def _score_kernel(x_hbm, batch_hbm, emb_hbm, cnt_hbm, scores_out,
                  xbuf, bbuf, embbuf, cntbuf, scorebuf):
  wid = lax.axis_index("c") * NS + lax.axis_index("s")
  g0 = wid * GQ + jnp.minimum(wid, GR)
  ng = GQ + jnp.where(wid < GR, 1, 0)
  nsc = lax.div(ng + (SCG - 1), SCG)

  pltpu.sync_copy(emb_hbm, embbuf)
  pltpu.sync_copy(cnt_hbm, cntbuf)

  iota = lax.iota(jnp.int32, LANES)
  iota_f = jnp.asarray(iota, jnp.float32)
  onehots = [
      jnp.maximum(1.0 - jnp.abs(iota_f - float(r)), 0.0) for r in range(LANES)]
  perms = [jnp.bitwise_xor(iota, 1 << k) for k in range(4)]

  def per_chunk(k, carry):
    rbase = pl.multiple_of(
        jnp.minimum(g0 * LANES + k * SCROWS, NUM_NODES - SCROWS), 16)
    pltpu.sync_copy(x_hbm.at[pl.ds(rbase, SCROWS)], xbuf)
    pltpu.sync_copy(batch_hbm.at[pl.ds(rbase, SCROWS)], bbuf)

    def per_group(j, carry):
      row0 = j * LANES
      bvec = bbuf[pl.ds(row0, LANES)]
      cv = plsc.load_gather(cntbuf, [bvec])
      svec = jnp.zeros((LANES,), jnp.float32)
      for r in range(LANES):
        gid = bvec[r]
        d = jnp.full((LANES,), -BIGF, jnp.float32)
        for c in range(FPR):
          xv = xbuf[row0 + r, pl.ds(c * LANES, LANES)]
          ev = embbuf[gid, pl.ds(c * LANES, LANES)]
          d = jnp.maximum(d, xv - ev)
        for p in perms:
          d = jnp.maximum(d, d[p])
        match = 1.0 - jnp.minimum(-d * BIGF * BIGF, 1.0)
        svec = svec + onehots[r] * match
      scorebuf[pl.ds(row0, LANES)] = svec / cv
      return carry

    lax.fori_loop(0, SCG, per_group, carry)
    pltpu.sync_copy(scorebuf, scores_out.at[pl.ds(rbase, SCROWS)])
    return carry

  lax.fori_loop(0, nsc, per_chunk, jnp.int32(0))


def _mesh():
  return plsc.VectorSubcoreMesh(
      core_axis_name="c", subcore_axis_name="s", num_cores=NC, num_subcores=NS)


@jax.jit
def kernel(x, batch):
  batch = batch.astype(jnp.int32)
  npad = (-NUM_NODES) % HIDDEN_DIM
  batch2d = jnp.concatenate(
      [batch, jnp.full((npad,), NUM_GRAPHS, jnp.int32)]
  ).reshape((NUM_NODES + npad) // HIDDEN_DIM, HIDDEN_DIM)

  seg = pl.kernel(
      _seg_kernel,
      out_type=(
          jax.ShapeDtypeStruct((NUM_GRAPHS, HIDDEN_DIM), jnp.float32),
          jax.ShapeDtypeStruct((NUM_GRAPHS,), jnp.float32),
      ),
      mesh=_mesh(),
      compiler_params=pltpu.CompilerParams(needs_layout_passes=False),
      scratch_types=[
          pltpu.VMEM((CH, HIDDEN_DIM), jnp.float32),
          pltpu.VMEM((GPW, HIDDEN_DIM), jnp.float32),
          pltpu.VMEM((GPW,), jnp.float32),
          pltpu.VMEM((LANES, HIDDEN_DIM), jnp.int32),
          pltpu.VMEM((LANES, HIDDEN_DIM), jnp.int32),
          pltpu.SemaphoreType.DMA,
          pltpu.SemaphoreType.DMA,
      ],
  )
  emb, cnt = seg(x, batch2d)

  score = pl.kernel(
      _score_kernel,
      out_type=jax.ShapeDtypeStruct((NUM_NODES,), jnp.float32),
      mesh=_mesh(),
      compiler_params=pltpu.CompilerParams(needs_layout_passes=False),
      scratch_types=[
          pltpu.VMEM((SCROWS, HIDDEN_DIM), jnp.float32),
          pltpu.VMEM((SCROWS,), jnp.int32),
          pltpu.VMEM((NUM_GRAPHS, HIDDEN_DIM), jnp.float32),
          pltpu.VMEM((NUM_GRAPHS,), jnp.float32),
          pltpu.VMEM((SCROWS,), jnp.float32),
      ],
  )
  scores = score(x, batch, emb, cnt)
  return (emb, scores)


# speedup vs baseline: 4.8116x; 2.6940x over previous
import jax
import jax.numpy as jnp
from jax import lax
from jax.experimental import pallas as pl
from jax.experimental.pallas import tpu as pltpu
from jax.experimental.pallas import tpu_sc as plsc

NUM_NODES = 100000
HIDDEN_DIM = 128
NUM_GRAPHS = 512

NC = 2
NS = 16
NW = NC * NS
GPW = NUM_GRAPHS // NW
LANES = 16
FPR = HIDDEN_DIM // LANES

CH = 240
EFF = CH - 8
NCAP = NUM_NODES - CH

NGROUPS = NUM_NODES // LANES
GQ, GR = divmod(NGROUPS, NW)
SCG = 8
SCROWS = SCG * LANES

NBLK = -(-NUM_NODES // HIDDEN_DIM)
NBLKR = 800

NEG_INF = float("-inf")
BIGF = 3.402823466e38


def _bcast(s):
  return lax.broadcast_in_dim(s, (LANES,), ())


def _seg_kernel(x_hbm, batch2d_hbm, coarse_hbm, emb_out, cnt_out,
                xbufA, xbufB, gmaxbuf, cntbuf, d1buf, d2buf, coarsebuf,
                sem1, sem2):
  wid = lax.axis_index("c") * NS + lax.axis_index("s")
  g_base = wid * GPW

  iota = lax.iota(jnp.int32, LANES)
  perms = [jnp.bitwise_xor(iota, 1 << k) for k in range(4)]

  iota_f = jnp.asarray(iota, jnp.float32)

  tgt1 = g_base + iota
  tgt2 = tgt1 + 1
  pltpu.sync_copy(coarse_hbm, coarsebuf)

  def bs_coarse(t, carry):
    lo1, hi1, lo2, hi2 = carry
    mid1 = lax.shift_right_arithmetic(lo1 + hi1, 1)
    mid2 = lax.shift_right_arithmetic(lo2 + hi2, 1)
    b1 = plsc.load_gather(coarsebuf, [jnp.minimum(mid1, NBLKR - 1)])
    b2 = plsc.load_gather(coarsebuf, [jnp.minimum(mid2, NBLKR - 1)])
    lo1 = jnp.where(b1 < tgt1, mid1 + 1, lo1)
    hi1 = jnp.where(b1 < tgt1, hi1, mid1)
    lo2 = jnp.where(b2 < tgt2, mid2 + 1, lo2)
    hi2 = jnp.where(b2 < tgt2, hi2, mid2)
    return (lo1, hi1, lo2, hi2)

  z = jnp.zeros((LANES,), jnp.int32)
  bfull = jnp.full((LANES,), NBLKR, jnp.int32)
  c1, _, c2, _ = lax.fori_loop(0, 10, bs_coarse, (z, bfull, z, bfull))
  blk1 = jnp.clip(jnp.minimum(c1, NBLKR) - 1, 0, NBLKR - 1)
  blk2 = jnp.clip(jnp.minimum(c2, NBLKR) - 1, 0, NBLKR - 1)

  cp1 = pltpu.async_copy(batch2d_hbm.at[blk1], d1buf, sem1)
  cp2 = pltpu.async_copy(batch2d_hbm.at[blk2], d2buf, sem2)
  cp1.wait()
  cp2.wait()

  def bs_fine(t, carry):
    lo1, hi1, lo2, hi2 = carry
    mid1 = lax.shift_right_arithmetic(lo1 + hi1, 1)
    mid2 = lax.shift_right_arithmetic(lo2 + hi2, 1)
    b1 = plsc.load_gather(d1buf, [iota, jnp.minimum(mid1, HIDDEN_DIM - 1)])
    b2 = plsc.load_gather(d2buf, [iota, jnp.minimum(mid2, HIDDEN_DIM - 1)])
    lo1 = jnp.where(b1 < tgt1, mid1 + 1, lo1)
    hi1 = jnp.where(b1 < tgt1, hi1, mid1)
    lo2 = jnp.where(b2 < tgt2, mid2 + 1, lo2)
    hi2 = jnp.where(b2 < tgt2, hi2, mid2)
    return (lo1, hi1, lo2, hi2)

  hfull = jnp.full((LANES,), HIDDEN_DIM, jnp.int32)
  f1, _, f2, _ = lax.fori_loop(0, 8, bs_fine, (z, hfull, z, hfull))
  starts_vec = jnp.minimum(blk1 * HIDDEN_DIM + jnp.minimum(f1, HIDDEN_DIM),
                           NUM_NODES)
  ends_vec = jnp.minimum(blk2 * HIDDEN_DIM + jnp.minimum(f2, HIDDEN_DIM),
                         NUM_NODES)

  onehots = [
      jnp.maximum(1.0 - jnp.abs(iota_f - float(r)), 0.0) for r in range(GPW)]

  def _ext(vec, i):
    return vec[_bcast(i)][0]

  def _base0(s):
    return pl.multiple_of(
        jnp.bitwise_and(jnp.minimum(s, NCAP), -8), 8)

  def _cp0(i, buf, sem):
    s = _ext(starts_vec, i)
    return pltpu.make_async_copy(x_hbm.at[pl.ds(_base0(s), CH)], buf, sem)

  _cp0(0, xbufA, sem1).start()
  cnt_vec = jnp.zeros((LANES,), jnp.float32)
  for i in range(GPW):
    xbuf, sem = (xbufA, sem1) if i % 2 == 0 else (xbufB, sem2)
    _cp0(i, xbuf, sem).wait()
    if i + 1 < GPW:
      _cp0(i + 1, xbufB if i % 2 == 0 else xbufA,
           sem2 if i % 2 == 0 else sem1).start()
    s = _ext(starts_vec, i)
    e = _ext(ends_vec, i)
    nk = lax.div(e - s + (EFF - 1), EFF)

    def p1_chunk(k, acc):
      pos = s + k * EFF
      base = _base0(pos)

      @pl.when(k > 0)
      def _():
        pltpu.sync_copy(x_hbm.at[pl.ds(base, CH)], xbuf)
      lo = pos - base
      hi = jnp.minimum(pos + EFF, e) - base

      def p1_row(j, acc):
        out = []
        for c in range(FPR):
          xv = xbuf[j, pl.ds(c * LANES, LANES)]
          out.append(jnp.maximum(acc[c], xv))
        return tuple(out)

      return lax.fori_loop(lo, hi, p1_row, acc)

    acc0 = tuple(jnp.full((LANES,), NEG_INF, jnp.float32) for _ in range(FPR))
    acc = lax.fori_loop(0, nk, p1_chunk, acc0)
    for c in range(FPR):
      gmaxbuf[i, pl.ds(c * LANES, LANES)] = acc[c]

    def p2_chunk(k, cnt):
      pos = s + k * EFF
      base = _base0(pos)
      @pl.when(jnp.logical_or(k > 0, nk > 1))
      def _():
        pltpu.sync_copy(x_hbm.at[pl.ds(base, CH)], xbuf)
      lo = pos - base
      hi = jnp.minimum(pos + EFF, e) - base

      def p2_row(j, cnt):
        d = jnp.full((LANES,), -BIGF, jnp.float32)
        for c in range(FPR):
          xv = xbuf[j, pl.ds(c * LANES, LANES)]
          d = jnp.maximum(d, xv - acc[c])
        for p in perms:
          d = jnp.maximum(d, d[p])
        return cnt + (1.0 - jnp.minimum(-d * BIGF * BIGF, 1.0))

      return lax.fori_loop(lo, hi, p2_row, cnt)

    cnt = lax.fori_loop(0, nk, p2_chunk, jnp.zeros((LANES,), jnp.float32))
    cnt = jnp.maximum(cnt, 1.0)
    cnt_vec = cnt_vec + onehots[i] * cnt

  cntbuf[...] = cnt_vec
  gb = pl.multiple_of(g_base, 8)
  pltpu.sync_copy(gmaxbuf, emb_out.at[pl.ds(gb, GPW)])
  pltpu.sync_copy(cntbuf, cnt_out.at[pl.ds(gb, GPW)])


def _score_kernel(x_hbm, batch_hbm, emb_hbm, cnt_hbm, scores_out,
                  xbufA, bbufA, xbufB, bbufB, embbuf, cntbuf, scorebuf,
                  semA, semB):
  wid = lax.axis_index("c") * NS + lax.axis_index("s")
  g0 = wid * GQ + jnp.minimum(wid, GR)

  pltpu.sync_copy(emb_hbm, embbuf)
  pltpu.sync_copy(cnt_hbm, cntbuf)

  iota = lax.iota(jnp.int32, LANES)
  iota_f = jnp.asarray(iota, jnp.float32)
  onehots = [
      jnp.maximum(1.0 - jnp.abs(iota_f - float(r)), 0.0) for r in range(LANES)]
  perms = [jnp.bitwise_xor(iota, 1 << k) for k in range(4)]

  def rbase_of(k):
    return pl.multiple_of(
        jnp.minimum(g0 * LANES + k * SCROWS, NUM_NODES - SCROWS), 16)

  def copies(k, xbuf, bbuf, sem):
    rbase = rbase_of(k)
    return (pltpu.make_async_copy(x_hbm.at[pl.ds(rbase, SCROWS)], xbuf, sem),
            pltpu.make_async_copy(batch_hbm.at[pl.ds(rbase, SCROWS)], bbuf,
                                  sem))

  def issue(k, xbuf, bbuf, sem):
    cx, cb = copies(k, xbuf, bbuf, sem)
    cx.start()
    cb.start()

  def wait(k, xbuf, bbuf, sem):
    cx, cb = copies(k, xbuf, bbuf, sem)
    cx.wait()
    cb.wait()

  def row_term(xbuf, row, ev):
    d = jnp.full((LANES,), -BIGF, jnp.float32)
    for c in range(FPR):
      xv = xbuf[row, pl.ds(c * LANES, LANES)]
      d = jnp.maximum(d, xv - ev[c])
    for p in perms:
      d = jnp.maximum(d, d[p])
    return 1.0 - jnp.minimum(-d * BIGF * BIGF, 1.0)

  def compute(k, xbuf, bbuf):
    rbase = rbase_of(k)

    def per_group(j, carry):
      row0 = j * LANES
      bvec = bbuf[pl.ds(row0, LANES)]
      cv = plsc.load_gather(cntbuf, [bvec])

      def fast():
        gid = bvec[0]
        ev = [embbuf[gid, pl.ds(c * LANES, LANES)] for c in range(FPR)]
        svec = jnp.zeros((LANES,), jnp.float32)
        for r in range(LANES):
          svec = svec + onehots[r] * row_term(xbuf, row0 + r, ev)
        return svec

      def slow():
        svec = jnp.zeros((LANES,), jnp.float32)
        for r in range(LANES):
          gid = bvec[r]
          ev = [embbuf[gid, pl.ds(c * LANES, LANES)] for c in range(FPR)]
          svec = svec + onehots[r] * row_term(xbuf, row0 + r, ev)
        return svec

      svec = lax.cond(bvec[0] == bvec[LANES - 1], fast, slow)
      scorebuf[pl.ds(row0, LANES)] = svec / cv
      return carry

    lax.fori_loop(0, SCG, per_group, jnp.int32(0))
    pltpu.sync_copy(scorebuf, scores_out.at[pl.ds(rbase, SCROWS)])

  NSC = -(-(GQ + 1) // SCG)
  issue(0, xbufA, bbufA, semA)

  def per_pair(p, carry):
    k = 2 * p
    issue(k + 1, xbufB, bbufB, semB)
    wait(k, xbufA, bbufA, semA)
    compute(k, xbufA, bbufA)
    issue(k + 2, xbufA, bbufA, semA)
    wait(k + 1, xbufB, bbufB, semB)
    compute(k + 1, xbufB, bbufB)
    return carry

  lax.fori_loop(0, (NSC + 1) // 2, per_pair, jnp.int32(0))
  wait(2 * ((NSC + 1) // 2), xbufA, bbufA, semA)


def _mesh():
  return plsc.VectorSubcoreMesh(
      core_axis_name="c", subcore_axis_name="s", num_cores=NC, num_subcores=NS)


@jax.jit
def kernel(x, batch):
  batch = batch.astype(jnp.int32)
  npad = (-NUM_NODES) % HIDDEN_DIM
  batch2d = jnp.concatenate(
      [batch, jnp.full((npad,), NUM_GRAPHS, jnp.int32)]
  ).reshape((NUM_NODES + npad) // HIDDEN_DIM, HIDDEN_DIM)
  coarse = jnp.concatenate(
      [batch2d[:, 0], jnp.full((NBLKR - batch2d.shape[0],), NUM_GRAPHS,
                               jnp.int32)])

  seg = pl.kernel(
      _seg_kernel,
      out_type=(
          jax.ShapeDtypeStruct((NUM_GRAPHS, HIDDEN_DIM), jnp.float32),
          jax.ShapeDtypeStruct((NUM_GRAPHS,), jnp.float32),
      ),
      mesh=_mesh(),
      compiler_params=pltpu.CompilerParams(needs_layout_passes=False),
      scratch_types=[
          pltpu.VMEM((CH, HIDDEN_DIM), jnp.float32),
          pltpu.VMEM((CH, HIDDEN_DIM), jnp.float32),
          pltpu.VMEM((GPW, HIDDEN_DIM), jnp.float32),
          pltpu.VMEM((GPW,), jnp.float32),
          pltpu.VMEM((LANES, HIDDEN_DIM), jnp.int32),
          pltpu.VMEM((LANES, HIDDEN_DIM), jnp.int32),
          pltpu.VMEM((NBLKR,), jnp.int32),
          pltpu.SemaphoreType.DMA,
          pltpu.SemaphoreType.DMA,
      ],
  )
  emb, cnt = seg(x, batch2d, coarse)

  score = pl.kernel(
      _score_kernel,
      out_type=jax.ShapeDtypeStruct((NUM_NODES,), jnp.float32),
      mesh=_mesh(),
      compiler_params=pltpu.CompilerParams(needs_layout_passes=False),
      scratch_types=[
          pltpu.VMEM((SCROWS, HIDDEN_DIM), jnp.float32),
          pltpu.VMEM((SCROWS,), jnp.int32),
          pltpu.VMEM((SCROWS, HIDDEN_DIM), jnp.float32),
          pltpu.VMEM((SCROWS,), jnp.int32),
          pltpu.VMEM((NUM_GRAPHS, HIDDEN_DIM), jnp.float32),
          pltpu.VMEM((NUM_GRAPHS,), jnp.float32),
          pltpu.VMEM((SCROWS,), jnp.float32),
          pltpu.SemaphoreType.DMA,
          pltpu.SemaphoreType.DMA,
      ],
  )
  scores = score(x, batch, emb, cnt)
  return (emb, scores)
